# baseline (speedup 1.0000x reference)
import functools

import jax
import jax.numpy as jnp
from jax import lax
from jax.experimental import pallas as pl
from jax.experimental.pallas import tpu as pltpu
from jax.experimental.pallas import tpu_sc as plsc

N_CONT = 13
N_CAT = 26
VOCAB = 100000
EDIM = 16
BATCH = 16384
N_IN = N_CONT + N_CAT
N_OUT = N_CAT * EDIM + N_CONT

_INFO = plsc.get_sparse_core_info()
_NC = _INFO.num_cores
_NS = _INFO.num_subcores
_L = _INFO.num_lanes
_NW = _NC * _NS

_PER_W = BATCH // _NW
_CB = 128
_NCHUNK = _PER_W // _CB
_ROWS = _CB * N_CAT
_GRP = _ROWS // _L


def _cat_emb_head(x_in_flat, table_flat):
  mesh = plsc.VectorSubcoreMesh(core_axis_name="c", subcore_axis_name="s")

  @functools.partial(
      pl.kernel,
      mesh=mesh,
      out_type=jax.ShapeDtypeStruct((BATCH * N_OUT,), jnp.float32),
      compiler_params=pltpu.CompilerParams(
          use_tc_tiling_on_sc=False, needs_layout_passes=False),
      scratch_types=[
          pltpu.VMEM((_CB * N_IN,), jnp.float32),
          pltpu.VMEM((_ROWS,), jnp.int32),
          pltpu.VMEM((_ROWS, EDIM), jnp.float32),
          pltpu.VMEM((_CB * N_OUT,), jnp.float32),
          pltpu.SemaphoreType.DMA,
      ],
  )
  def k(xin_hbm, table_hbm, out_hbm, xbuf, idxbuf, rowbuf, outbuf, sem):
    wid = lax.axis_index("s") * _NC + lax.axis_index("c")
    base_w = wid * _PER_W
    lane = lax.iota(jnp.int32, _L)
    contmask = lane < N_CONT

    def chunk_body(c, carry):
      r0 = pl.multiple_of(base_w + c * _CB, _CB)
      pltpu.sync_copy(xin_hbm.at[pl.ds(r0 * N_IN, _CB * N_IN)], xbuf)

      def grp_body(g, carry2):
        off = pl.multiple_of(g * _L, _L)
        p = off + lane
        b = lax.div(p, N_CAT)
        i = lax.rem(p, N_CAT)
        src = b * N_IN + (N_CONT + i)
        v = plsc.load_gather(xbuf, [src]).astype(jnp.int32)
        idxbuf[pl.ds(off, _L)] = v + i * VOCAB
        return carry2

      lax.fori_loop(0, _GRP, grp_body, 0, unroll=4)

      pltpu.async_copy(table_hbm.at[idxbuf], rowbuf, sem).wait()

      def row_body(b, carry3):
        dst0 = b * N_OUT + lane
        for i in range(N_CAT):
          v = rowbuf[b * N_CAT + i]
          plsc.store_scatter(outbuf, [dst0 + i * EDIM], v)
        cont = plsc.load_gather(xbuf, [b * N_IN + lane])
        plsc.store_scatter(outbuf, [dst0 + N_CAT * EDIM], cont, mask=contmask)
        return carry3

      lax.fori_loop(0, _CB, row_body, 0, unroll=2)

      pltpu.sync_copy(outbuf, out_hbm.at[pl.ds(r0 * N_OUT, _CB * N_OUT)])
      return carry

    lax.fori_loop(0, _NCHUNK, chunk_body, 0)

  return k(x_in_flat, table_flat)


def kernel(x_in, tables):
  x_in_flat = x_in.reshape(-1)
  table_flat = tables.reshape(N_CAT * VOCAB, EDIM)
  out_flat = _cat_emb_head(x_in_flat, table_flat)
  return out_flat.reshape(BATCH, N_OUT)
